# Initial kernel scaffold; baseline (speedup 1.0000x reference)
import jax
import jax.numpy as jnp
from jax.experimental import pallas as pl


def kernel(atom_fea, atom_symm, nbr_fea, nbr_idx, crystal_atom_idx, params):
    raise NotImplementedError("write your pallas kernel here")


# speedup vs baseline: 1.6910x; 1.6910x over previous
import functools

import jax
import jax.numpy as jnp
from jax import lax
from jax.experimental import pallas as pl
from jax.experimental.pallas import tpu as pltpu
from jax.experimental.pallas import tpu_sc as plsc

F = 64
EPS = 1e-5


def _sc_gather(table, idx_flat):
    n_rows = table.shape[0]
    feat = table.shape[1]
    b_tot = idx_flat.shape[0]
    info = plsc.get_sparse_core_info()
    nw = info.num_cores * info.num_subcores
    b_per_w = b_tot // nw
    assert b_per_w * nw == b_tot
    ch = 40
    grp = 5
    nch = b_per_w // ch
    ngrp = nch // grp
    assert ch * nch == b_per_w and grp * ngrp == nch
    idx3 = idx_flat.reshape(nw, nch, ch)

    mesh = plsc.VectorSubcoreMesh(core_axis_name="c", subcore_axis_name="s")

    @functools.partial(
        pl.kernel,
        mesh=mesh,
        out_type=jax.ShapeDtypeStruct((b_tot, feat), jnp.float32),
        scratch_types=[
            pltpu.VMEM((nch, ch), jnp.int32),
            pltpu.VMEM((grp, ch, feat), jnp.float32),
            pltpu.SemaphoreType.DMA,
            pltpu.SemaphoreType.DMA,
        ],
    )
    def gather_k(table_hbm, idx_hbm, out_hbm, idx_v, buf, gsem, wsem):
        wid = lax.axis_index("s") * info.num_cores + lax.axis_index("c")
        base = wid * b_per_w
        pltpu.sync_copy(idx_hbm.at[wid], idx_v)

        def group(g, carry):
            c0 = grp * g
            hs = [
                pltpu.async_copy(table_hbm.at[idx_v.at[c0 + b]], buf.at[b],
                                 gsem)
                for b in range(grp)
            ]
            ws = []
            for b in range(grp):
                hs[b].wait()
                ws.append(
                    pltpu.async_copy(
                        buf.at[b],
                        out_hbm.at[pl.ds(base + (c0 + b) * ch, ch)], wsem))
            for w in ws:
                w.wait()
            return carry

        lax.fori_loop(0, ngrp, group, 0)

    return gather_k(table, idx3)


def _emb_call(atom_fea, w, b):
    n, orig = atom_fea.shape
    bn = 2000
    grid = n // bn

    def body(x_ref, w_ref, b_ref, o_ref):
        y = (jnp.dot(x_ref[...], w_ref[...], preferred_element_type=jnp.float32)
             + b_ref[...])
        o_ref[...] = jnp.concatenate([y, jnp.zeros_like(y)], axis=1)

    return pl.pallas_call(
        body,
        grid=(grid,),
        in_specs=[
            pl.BlockSpec((bn, orig), lambda i: (i, 0)),
            pl.BlockSpec((orig, F), lambda i: (0, 0)),
            pl.BlockSpec((1, F), lambda i: (0, 0)),
        ],
        out_specs=pl.BlockSpec((bn, 2 * F), lambda i: (i, 0)),
        out_shape=jax.ShapeDtypeStruct((n, 2 * F), jnp.float32),
    )(atom_fea, w, b.reshape(1, F))


def _gate_halves(x, an, nf, wts, bn, m):
    wsf, wsc, wnbf, wnbc, wff, wfc, bf, bc = wts
    ps_f = jnp.dot(x, wsf, preferred_element_type=jnp.float32) + bf
    ps_c = jnp.dot(x, wsc, preferred_element_type=jnp.float32) + bc
    ef = (jnp.dot(an, wnbf, preferred_element_type=jnp.float32)
          + jnp.dot(nf, wff, preferred_element_type=jnp.float32))
    ec = (jnp.dot(an, wnbc, preferred_element_type=jnp.float32)
          + jnp.dot(nf, wfc, preferred_element_type=jnp.float32))
    gf = ef.reshape(bn, m, F) + ps_f[:, None, :]
    gc = ec.reshape(bn, m, F) + ps_c[:, None, :]
    return gf, gc


def _conv_pass1(x, an, nf, wts, bn, m):
    n = x.shape[0]
    grid = n // bn
    nbr = nf.shape[1]

    def body(x_ref, an_ref, nf_ref, wsf, wsc, wnbf, wnbc, wff, wfc, bf, bc,
             st_ref):
        i = pl.program_id(0)
        wts_v = (wsf[...], wsc[...], wnbf[...], wnbc[...], wff[...], wfc[...],
                 bf[...], bc[...])
        gf, gc = _gate_halves(x_ref[:, :F], an_ref[:, :F], nf_ref[...], wts_v,
                              bn, m)
        sf = jnp.sum(jnp.sum(gf, axis=1), axis=0, keepdims=True)
        sc_ = jnp.sum(jnp.sum(gc, axis=1), axis=0, keepdims=True)
        qf = jnp.sum(jnp.sum(gf * gf, axis=1), axis=0, keepdims=True)
        qc = jnp.sum(jnp.sum(gc * gc, axis=1), axis=0, keepdims=True)
        st = jnp.concatenate([sf, sc_, qf, qc], axis=0)

        @pl.when(i == 0)
        def _():
            st_ref[...] = jnp.zeros_like(st_ref)

        st_ref[...] += st

    wspec = pl.BlockSpec((F, F), lambda i: (0, 0))
    nspec = pl.BlockSpec((nbr, F), lambda i: (0, 0))
    bspec = pl.BlockSpec((1, F), lambda i: (0, 0))
    return pl.pallas_call(
        body,
        grid=(grid,),
        in_specs=[
            pl.BlockSpec((bn, 2 * F), lambda i: (i, 0)),
            pl.BlockSpec((bn * m, 2 * F), lambda i: (i, 0)),
            pl.BlockSpec((bn * m, nbr), lambda i: (i, 0)),
            wspec, wspec, wspec, wspec, nspec, nspec, bspec, bspec,
        ],
        out_specs=pl.BlockSpec((4, F), lambda i: (0, 0)),
        out_shape=jax.ShapeDtypeStruct((4, F), jnp.float32),
    )(x, an, nf.reshape(-1, nbr), *wts[:6],
      wts[6].reshape(1, F), wts[7].reshape(1, F))


def _conv_pass2(x, an, nf, wts, scsh, bn, m):
    n = x.shape[0]
    grid = n // bn
    nbr = nf.shape[1]

    def body(x_ref, an_ref, nf_ref, wsf, wsc, wnbf, wnbc, wff, wfc, bf, bc,
             ss_ref, ns_ref, st_ref):
        i = pl.program_id(0)
        wts_v = (wsf[...], wsc[...], wnbf[...], wnbc[...], wff[...], wfc[...],
                 bf[...], bc[...])
        gf, gc = _gate_halves(x_ref[:, :F], an_ref[:, :F], nf_ref[...], wts_v,
                              bn, m)
        ss = ss_ref[...]
        gf = gf * ss[0:1][:, None, :] + ss[1:2][:, None, :]
        gc = gc * ss[2:3][:, None, :] + ss[3:4][:, None, :]
        z = jax.nn.sigmoid(gf) * jax.nn.softplus(gc)
        ns = jnp.sum(z, axis=1)
        ns_ref[...] = ns
        s = jnp.sum(ns, axis=0, keepdims=True)
        q = jnp.sum(ns * ns, axis=0, keepdims=True)
        st = jnp.concatenate([s, q], axis=0)

        @pl.when(i == 0)
        def _():
            st_ref[...] = jnp.zeros_like(st_ref)

        st_ref[...] += st

    wspec = pl.BlockSpec((F, F), lambda i: (0, 0))
    nspec = pl.BlockSpec((nbr, F), lambda i: (0, 0))
    bspec = pl.BlockSpec((1, F), lambda i: (0, 0))
    return pl.pallas_call(
        body,
        grid=(grid,),
        in_specs=[
            pl.BlockSpec((bn, 2 * F), lambda i: (i, 0)),
            pl.BlockSpec((bn * m, 2 * F), lambda i: (i, 0)),
            pl.BlockSpec((bn * m, nbr), lambda i: (i, 0)),
            wspec, wspec, wspec, wspec, nspec, nspec, bspec, bspec,
            pl.BlockSpec((4, F), lambda i: (0, 0)),
        ],
        out_specs=[
            pl.BlockSpec((bn, F), lambda i: (i, 0)),
            pl.BlockSpec((2, F), lambda i: (0, 0)),
        ],
        out_shape=[
            jax.ShapeDtypeStruct((n, F), jnp.float32),
            jax.ShapeDtypeStruct((2, F), jnp.float32),
        ],
    )(x, an, nf.reshape(-1, nbr), *wts[:6],
      wts[6].reshape(1, F), wts[7].reshape(1, F), scsh)


def _conv_pass3(x, ns, scsh2):
    n = x.shape[0]
    bn = 2000
    grid = n // bn

    def body(x_ref, ns_ref, ss_ref, o_ref):
        ss = ss_ref[...]
        y = jax.nn.softplus(x_ref[:, :F] + ns_ref[...] * ss[0:1] + ss[1:2])
        o_ref[...] = jnp.concatenate([y, jnp.zeros_like(y)], axis=1)

    return pl.pallas_call(
        body,
        grid=(grid,),
        in_specs=[
            pl.BlockSpec((bn, 2 * F), lambda i: (i, 0)),
            pl.BlockSpec((bn, F), lambda i: (i, 0)),
            pl.BlockSpec((2, F), lambda i: (0, 0)),
        ],
        out_specs=pl.BlockSpec((bn, 2 * F), lambda i: (i, 0)),
        out_shape=jax.ShapeDtypeStruct((n, 2 * F), jnp.float32),
    )(x, ns, scsh2)


def _head_call(r_mat, symm_row, x, w_fc, b_fc, w_out_pad, b_out_pad):
    c = r_mat.shape[0]
    h = w_fc.shape[1]
    po = w_out_pad.shape[1]

    def body(r_ref, sy_ref, x_ref, wfc_ref, bfc_ref, wo_ref, bo_ref, o_ref):
        rw = r_ref[...] * jnp.abs(sy_ref[...])
        denom = jnp.sum(rw, axis=1, keepdims=True)
        crys = jnp.dot(rw, x_ref[:, :F], preferred_element_type=jnp.float32)
        crys = jax.nn.softplus(crys / denom)
        hid = jax.nn.softplus(
            jnp.dot(crys, wfc_ref[...], preferred_element_type=jnp.float32)
            + bfc_ref[...])
        o_ref[...] = (jnp.dot(hid, wo_ref[...],
                              preferred_element_type=jnp.float32) + bo_ref[...])

    n = x.shape[0]
    return pl.pallas_call(
        body,
        grid=(1,),
        in_specs=[
            pl.BlockSpec((c, n), lambda i: (0, 0)),
            pl.BlockSpec((1, n), lambda i: (0, 0)),
            pl.BlockSpec((n, 2 * F), lambda i: (0, 0)),
            pl.BlockSpec((F, h), lambda i: (0, 0)),
            pl.BlockSpec((1, h), lambda i: (0, 0)),
            pl.BlockSpec((h, po), lambda i: (0, 0)),
            pl.BlockSpec((1, po), lambda i: (0, 0)),
        ],
        out_specs=pl.BlockSpec((c, po), lambda i: (0, 0)),
        out_shape=jax.ShapeDtypeStruct((c, po), jnp.float32),
    )(r_mat, symm_row, x, w_fc, b_fc, w_out_pad, b_out_pad)


def _bn_scale_shift(s, q, count, g, be):
    mean = s / count
    var = q / count - mean * mean
    scale = g / jnp.sqrt(var + EPS)
    shift = be - mean * scale
    return scale, shift


def kernel(atom_fea, atom_symm, nbr_fea, nbr_idx, crystal_atom_idx, params):
    n, m = nbr_idx.shape
    nbr = nbr_fea.shape[2]
    bn = 200
    idx_flat = nbr_idx.reshape(-1).astype(jnp.int32)
    nf_flat = nbr_fea.reshape(n * m, nbr)

    x = _emb_call(atom_fea, params["W_emb"], params["b_emb"])

    for cparams in params["convs"]:
        wfull = cparams["W_full"]
        wts = (
            wfull[0:F, 0:F], wfull[0:F, F:2 * F],
            wfull[F:2 * F, 0:F], wfull[F:2 * F, F:2 * F],
            wfull[2 * F:, 0:F], wfull[2 * F:, F:2 * F],
            cparams["b_full"][0:F], cparams["b_full"][F:2 * F],
        )
        an = _sc_gather(x, idx_flat)
        st = _conv_pass1(x, an, nf_flat, wts, bn, m)
        cnt = jnp.float32(n * m)
        g1 = cparams["g1"]
        be1 = cparams["be1"]
        sc_f, sh_f = _bn_scale_shift(st[0], st[2], cnt, g1[0:F], be1[0:F])
        sc_c, sh_c = _bn_scale_shift(st[1], st[3], cnt, g1[F:], be1[F:])
        scsh = jnp.stack([sc_f, sh_f, sc_c, sh_c])
        ns, st2 = _conv_pass2(x, an, nf_flat, wts, scsh, bn, m)
        sc2, sh2 = _bn_scale_shift(st2[0], st2[1], jnp.float32(n),
                                   cparams["g2"], cparams["be2"])
        x = _conv_pass3(x, ns, jnp.stack([sc2, sh2]))

    c = crystal_atom_idx.shape[0]
    r_mat = jnp.zeros((c, n), jnp.float32).at[
        jnp.arange(c)[:, None], crystal_atom_idx.astype(jnp.int32)
    ].set(1.0)
    w_out_pad = jnp.pad(params["W_out"], ((0, 0), (0, 7)))
    b_out_pad = jnp.pad(params["b_out"], ((0, 7))).reshape(1, 8)
    out = _head_call(r_mat, jnp.abs(atom_symm).reshape(1, n), x,
                     params["W_fc"], params["b_fc"].reshape(1, -1),
                     w_out_pad, b_out_pad)
    return out[:, 0:1]
